# baseline (speedup 1.0000x reference)
import functools

import jax
import jax.numpy as jnp
from jax import lax
from jax.experimental import pallas as pl
from jax.experimental.pallas import tpu as pltpu
from jax.experimental.pallas import tpu_sc as plsc

TOTAL = 262144
NR = 4096
NW = 32
CHUNK = TOTAL // NW
NVEC = CHUNK // 16
RPW = NR // NW

_f32 = jnp.float32
_i32 = jnp.int32


def _iota16():
    return lax.iota(_i32, 16)


def _mesh():
    return plsc.VectorSubcoreMesh(core_axis_name="c", subcore_axis_name="s")


_CPARAMS = pltpu.CompilerParams(needs_layout_passes=False)


def _wid():
    return lax.axis_index("c") * 16 + lax.axis_index("s")


def _l1_body(sig_hbm, dt_hbm, seg_hbm, s_hbm, l_hbm, praw_hbm, stats_hbm,
             sig_v, dt_v, s_v, l_v, seg_v, praw_v, stage_v):
    wid = _wid()
    base = wid * CHUNK
    iota = _iota16()

    seg_v[pl.ds(0, 16)] = jnp.full((16,), -1, _i32)
    @pl.when(wid > 0)
    def _():
        pltpu.sync_copy(seg_hbm.at[pl.ds(base - 8, 8)], seg_v.at[pl.ds(0, 8)])
    pltpu.sync_copy(seg_hbm.at[pl.ds(base, CHUNK)], seg_v.at[pl.ds(8, CHUNK)])
    pltpu.sync_copy(sig_hbm.at[pl.ds(base, CHUNK)], sig_v)
    pltpu.sync_copy(dt_hbm.at[pl.ds(base, CHUNK)], dt_v)

    def zbody(i, _):
        praw_v[pl.ds(i * 16, 16)] = jnp.zeros((16,), _f32)
        return 0
    lax.fori_loop(0, NR // 16, zbody, 0)

    zeros = jnp.zeros((16,), _f32)

    def body(v, carry):
        csum, hsv, slv = carry
        off = v * 16
        s16 = sig_v[pl.ds(off, 16)] * dt_v[pl.ds(off, 16)]
        s_v[pl.ds(off, 16)] = s16
        incl = plsc.cumsum(s16)
        lx = csum + (incl - s16)
        l_v[pl.ds(off, 16)] = lx
        segc = seg_v[pl.ds(off + 8, 16)]
        segp = seg_v[pl.ds(off + 7, 16)]
        rs = segc != segp
        plsc.store_scatter(praw_v, [segc], lx, mask=rs)
        csum = csum + plsc.cummax(lax.rev(incl, (0,)))
        hsv = jnp.maximum(hsv, rs.astype(_f32))
        slv = jnp.maximum(slv, jnp.where(rs, lx, _f32(-1.0)))
        return csum, hsv, slv

    tot_v, hsv, slv = lax.fori_loop(
        0, NVEC, body, (zeros, zeros, zeros - 1.0))
    tot = tot_v[0]
    hs = jnp.max(hsv)
    slast = jnp.max(slv)

    pltpu.sync_copy(s_v, s_hbm.at[pl.ds(base, CHUNK)])
    pltpu.sync_copy(l_v, l_hbm.at[pl.ds(base, CHUNK)])
    pltpu.sync_copy(praw_v, praw_hbm.at[wid])

    head = seg_v[pl.ds(0, 16)]
    tail = seg_v[pl.ds(CHUNK - 8, 16)]
    pos0 = (head[7] != head[8]).astype(_f32)
    stats = (jnp.where(iota == 0, tot, 0.0)
             + jnp.where(iota == 1, hs, 0.0)
             + jnp.where(iota == 2, slast, 0.0)
             + jnp.where(iota == 3, pos0, 0.0)
             + jnp.where(iota == 4, head[8].astype(_f32), 0.0)
             + jnp.where(iota == 5, tail[15].astype(_f32), 0.0)).astype(_f32)
    stage_v[...] = stats
    pltpu.sync_copy(stage_v, stats_hbm.at[pl.ds(wid * 16, 16)])


def _l2_body(s_hbm, l_hbm, seg_hbm, rgb_hbm, z_hbm, praw_hbm, stats_hbm,
             w_hbm, acc_hbm,
             s_v, l_v, z_v, w_v, rgb_v, seg_v, praw_v, stats_v, acc_v, ex_s):
    wid = _wid()
    base = wid * CHUNK
    iota = _iota16()

    seg_v[pl.ds(0, 16)] = jnp.full((16,), -1, _i32)
    @pl.when(wid > 0)
    def _():
        pltpu.sync_copy(seg_hbm.at[pl.ds(base - 8, 8)], seg_v.at[pl.ds(0, 8)])
    pltpu.sync_copy(seg_hbm.at[pl.ds(base, CHUNK)], seg_v.at[pl.ds(8, CHUNK)])
    seg_v[pl.ds(8 + CHUNK, 16)] = jnp.full((16,), -2, _i32)
    @pl.when(wid < NW - 1)
    def _():
        pltpu.sync_copy(seg_hbm.at[pl.ds(base + CHUNK, 8)],
                        seg_v.at[pl.ds(8 + CHUNK, 8)])
    pltpu.sync_copy(s_hbm.at[pl.ds(base, CHUNK)], s_v)
    pltpu.sync_copy(l_hbm.at[pl.ds(base, CHUNK)], l_v)
    pltpu.sync_copy(z_hbm.at[pl.ds(base, CHUNK)], z_v)
    pltpu.sync_copy(rgb_hbm.at[pl.ds(base * 3, CHUNK * 3)], rgb_v)
    pltpu.sync_copy(praw_hbm.at[wid], praw_v)
    pltpu.sync_copy(stats_hbm, stats_v)

    r0 = iota * 16
    r1 = (iota + 16) * 16
    tot0 = plsc.load_gather(stats_v, [r0])
    tot1 = plsc.load_gather(stats_v, [r1])
    hs0 = plsc.load_gather(stats_v, [r0 + 1])
    hs1 = plsc.load_gather(stats_v, [r1 + 1])
    sl0 = plsc.load_gather(stats_v, [r0 + 2])
    sl1 = plsc.load_gather(stats_v, [r1 + 2])

    bw = (jnp.sum(jnp.where(iota < wid, tot0, 0.0))
          + jnp.sum(jnp.where(iota + 16 < wid, tot1, 0.0)))

    head = seg_v[pl.ds(0, 16)]
    pos0_is_start = head[7] != head[8]
    first_id = head[8]

    @pl.when(jnp.logical_not(pos0_is_start))
    def _():
        c0 = jnp.where((hs0 > 0.5) & (iota < wid), iota, -1)
        c1 = jnp.where((hs1 > 0.5) & (iota + 16 < wid), iota + 16, -1)
        wstar = jnp.maximum(jnp.max(c0), jnp.max(c1))
        bws = (jnp.sum(jnp.where(iota < wstar, tot0, 0.0))
               + jnp.sum(jnp.where(iota + 16 < wstar, tot1, 0.0)))
        slws = (jnp.sum(jnp.where(iota == wstar, sl0, 0.0))
                + jnp.sum(jnp.where(iota + 16 == wstar, sl1, 0.0)))
        ov = (bws + slws) - bw
        plsc.store_scatter(praw_v, [jnp.full((16,), first_id, _i32)],
                           jnp.full((16,), ov, _f32), mask=iota == 0)

    def zbody(i, _):
        acc_v[pl.ds(i * 16, 16)] = jnp.zeros((16,), _f32)
        return 0
    lax.fori_loop(0, (NR * 8) // 16, zbody, 0)

    def body(v, carries):
        off = v * 16
        s16 = s_v[pl.ds(off, 16)]
        l16 = l_v[pl.ds(off, 16)]
        z16 = z_v[pl.ds(off, 16)]
        segc = seg_v[pl.ds(off + 8, 16)]
        segp = seg_v[pl.ds(off + 7, 16)]
        segn = seg_v[pl.ds(off + 9, 16)]
        idx3 = (off + iota) * 3
        rr = plsc.load_gather(rgb_v, [idx3])
        gg = plsc.load_gather(rgb_v, [idx3 + 1])
        bb = plsc.load_gather(rgb_v, [idx3 + 2])

        pv = plsc.load_gather(praw_v, [segc])
        excl = l16 - pv
        trans = jnp.exp(-excl)
        alpha = 1.0 - jnp.exp(-s16)
        wg = trans * alpha
        w_v[pl.ds(off, 16)] = wg

        rs = segc != segp
        re = segc != segn
        s_lane1 = plsc.cummax(jnp.where(rs, iota + 1, 0))
        fr = s_lane1 == 0
        sidx = jnp.maximum(s_lane1 - 1, 0)
        fr15v = plsc.cummax(lax.rev(s_lane1, (0,))) == 0

        outs = []
        for c, (q, (cb, cstart)) in enumerate(
                zip((s16, wg * rr, wg * gg, wg * bb, wg * z16), carries)):
            inclq = plsc.cumsum(q)
            exq = inclq - q
            ex_s[pl.ds(c * 16, 16)] = exq
            sxg = plsc.load_gather(ex_s, [sidx + c * 16])
            startex = jnp.where(fr, 0.0, sxg)
            cstart_abs = jnp.where(fr, cstart, cb + sxg)
            runsum = (cb + inclq) - cstart_abs
            plsc.addupdate_scatter(acc_v, [segc * 8 + c], runsum, mask=re)
            sx15 = plsc.cummax(lax.rev(startex, (0,)))
            cstart_n = jnp.where(fr15v, cstart, cb + sx15)
            cb_n = cb + plsc.cummax(lax.rev(inclq, (0,)))
            outs.append((cb_n, cstart_n))
        return tuple(outs)

    zv = jnp.zeros((16,), _f32)
    fcarries = lax.fori_loop(
        0, NVEC, body, ((zv, zv),) * 5)

    tail = seg_v[pl.ds(CHUNK - 8, 16)]
    last_id = tail[15]
    lane0 = _iota16() == 0
    for c, (cb, cstart) in enumerate(fcarries):
        fidx = jnp.full((16,), last_id * 8 + c, _i32)
        plsc.addupdate_scatter(acc_v, [fidx], cb - cstart, mask=lane0)

    pltpu.sync_copy(w_v, w_hbm.at[pl.ds(base, CHUNK)])
    pltpu.sync_copy(acc_v, acc_hbm.at[wid])


def _l3_body(acc_hbm, tex_hbm, flag_hbm, stats_hbm,
             rgb_hbm, dep_hbm, bg_hbm,
             tmp_v, accsum_v, tex_v, flag_v, rgb_s, dep_s, bg_s, stats_v):
    wid = _wid()
    r0 = wid * RPW
    iota = _iota16()

    for j in range(5 * RPW // 16):
        accsum_v[pl.ds(j * 16, 16)] = jnp.zeros((16,), _f32)

    pltpu.sync_copy(tex_hbm.at[pl.ds(r0, RPW)], tex_v)
    pltpu.sync_copy(flag_hbm, flag_v)
    pltpu.sync_copy(stats_hbm, stats_v)

    mylo = (wid * RPW).astype(_f32)
    myhi = mylo + _f32(RPW - 1)

    def body(k, _):
        srow = stats_v[pl.ds(k * 16, 16)]
        fid = srow[4]
        lid = srow[5]

        @pl.when((fid <= myhi) & (lid >= mylo))
        def _():
            pltpu.sync_copy(acc_hbm.at[k, pl.ds(r0 * 8, RPW * 8)], tmp_v)
            for c in range(5):
                for g in range(RPW // 16):
                    idx = (iota + g * 16) * 8 + c
                    val = plsc.load_gather(tmp_v, [idx])
                    sl = pl.ds(c * RPW + g * 16, 16)
                    accsum_v[sl] = accsum_v[sl] + val
        return 0

    lax.fori_loop(0, NW, body, 0)

    flag = flag_v[pl.ds(0, 16)]
    for g in range(RPW // 16):
        sl = pl.ds(g * 16, 16)
        tot = accsum_v[pl.ds(0 * RPW + g * 16, 16)]
        wr = accsum_v[pl.ds(1 * RPW + g * 16, 16)]
        wgr = accsum_v[pl.ds(2 * RPW + g * 16, 16)]
        wb = accsum_v[pl.ds(3 * RPW + g * 16, 16)]
        wz = accsum_v[pl.ds(4 * RPW + g * 16, 16)]
        bg = jnp.exp(-tot)
        dep = wz + flag * bg * tex_v[sl]
        bg_s[sl] = bg
        dep_s[sl] = dep
        ridx = (iota + g * 16) * 3
        plsc.store_scatter(rgb_s, [ridx], wr)
        plsc.store_scatter(rgb_s, [ridx + 1], wgr)
        plsc.store_scatter(rgb_s, [ridx + 2], wb)

    pltpu.sync_copy(bg_s, bg_hbm.at[pl.ds(r0, RPW)])
    pltpu.sync_copy(dep_s, dep_hbm.at[pl.ds(r0, RPW)])
    pltpu.sync_copy(rgb_s, rgb_hbm.at[pl.ds(r0 * 3, RPW * 3)])


def kernel(rgb_samples, radiance_samples, ray_samples_z, ray_samples_dt,
           ray_t_exit, segment_ids, use_ray_t_exit):
    sigma = radiance_samples.reshape(TOTAL)
    rgb_flat = rgb_samples.reshape(TOTAL * 3)
    tex = ray_t_exit.reshape(NR)
    seg = segment_ids.astype(_i32)
    flag16 = jnp.broadcast_to(
        (use_ray_t_exit != 0).astype(_f32), (16,))

    mesh = _mesh()

    l1 = functools.partial(
        pl.kernel,
        out_type=(
            jax.ShapeDtypeStruct((TOTAL,), _f32),
            jax.ShapeDtypeStruct((TOTAL,), _f32),
            jax.ShapeDtypeStruct((NW, NR), _f32),
            jax.ShapeDtypeStruct((NW * 16,), _f32),
        ),
        mesh=mesh,
        compiler_params=_CPARAMS,
        scratch_types=[
            pltpu.VMEM((CHUNK,), _f32),
            pltpu.VMEM((CHUNK,), _f32),
            pltpu.VMEM((CHUNK,), _f32),
            pltpu.VMEM((CHUNK,), _f32),
            pltpu.VMEM((CHUNK + 8,), _i32),
            pltpu.VMEM((NR,), _f32),
            pltpu.VMEM((16,), _f32),
        ],
    )(_l1_body)
    s_arr, l_arr, praw, stats = l1(sigma, ray_samples_dt, seg)

    l2 = functools.partial(
        pl.kernel,
        out_type=(
            jax.ShapeDtypeStruct((TOTAL,), _f32),
            jax.ShapeDtypeStruct((NW, NR * 8), _f32),
        ),
        mesh=mesh,
        compiler_params=_CPARAMS,
        scratch_types=[
            pltpu.VMEM((CHUNK,), _f32),
            pltpu.VMEM((CHUNK,), _f32),
            pltpu.VMEM((CHUNK,), _f32),
            pltpu.VMEM((CHUNK,), _f32),
            pltpu.VMEM((CHUNK * 3,), _f32),
            pltpu.VMEM((CHUNK + 24,), _i32),
            pltpu.VMEM((NR,), _f32),
            pltpu.VMEM((NW * 16,), _f32),
            pltpu.VMEM((NR * 8,), _f32),
            pltpu.VMEM((5 * 16,), _f32),
        ],
    )(_l2_body)
    w_arr, acc = l2(s_arr, l_arr, seg, rgb_flat, ray_samples_z, praw, stats)

    l3 = functools.partial(
        pl.kernel,
        out_type=(
            jax.ShapeDtypeStruct((NR * 3,), _f32),
            jax.ShapeDtypeStruct((NR,), _f32),
            jax.ShapeDtypeStruct((NR,), _f32),
        ),
        mesh=mesh,
        compiler_params=_CPARAMS,
        scratch_types=[
            pltpu.VMEM((RPW * 8,), _f32),
            pltpu.VMEM((5 * RPW,), _f32),
            pltpu.VMEM((RPW,), _f32),
            pltpu.VMEM((16,), _f32),
            pltpu.VMEM((RPW * 3,), _f32),
            pltpu.VMEM((RPW,), _f32),
            pltpu.VMEM((RPW,), _f32),
            pltpu.VMEM((NW * 16,), _f32),
        ],
    )(_l3_body)
    rgb_out, dep, bg = l3(acc, tex, flag16, stats)

    return (rgb_out.reshape(NR, 3), dep.reshape(NR, 1), bg.reshape(NR, 1),
            w_arr.reshape(TOTAL, 1))


# speedup vs baseline: 1.0238x; 1.0238x over previous
import functools

import jax
import jax.numpy as jnp
from jax import lax
from jax.experimental import pallas as pl
from jax.experimental.pallas import tpu as pltpu
from jax.experimental.pallas import tpu_sc as plsc

TOTAL = 262144
NR = 4096
NW = 32
CHUNK = TOTAL // NW
NVEC = CHUNK // 16
RPW = NR // NW

_f32 = jnp.float32
_i32 = jnp.int32


def _iota16():
    return lax.iota(_i32, 16)


def _mesh():
    return plsc.VectorSubcoreMesh(core_axis_name="c", subcore_axis_name="s")


_CPARAMS = pltpu.CompilerParams(needs_layout_passes=False)


def _wid():
    return lax.axis_index("c") * 16 + lax.axis_index("s")


def _l1_body(sig_hbm, dt_hbm, seg_hbm, s_hbm, l_hbm, praw_hbm, stats_hbm,
             sig_v, dt_v, s_v, l_v, seg_v, praw_v, stage_v):
    wid = _wid()
    base = wid * CHUNK
    iota = _iota16()

    seg_v[pl.ds(0, 16)] = jnp.full((16,), -1, _i32)
    @pl.when(wid > 0)
    def _():
        pltpu.sync_copy(seg_hbm.at[pl.ds(base - 8, 8)], seg_v.at[pl.ds(0, 8)])
    pltpu.sync_copy(seg_hbm.at[pl.ds(base, CHUNK)], seg_v.at[pl.ds(8, CHUNK)])
    pltpu.sync_copy(sig_hbm.at[pl.ds(base, CHUNK)], sig_v)
    pltpu.sync_copy(dt_hbm.at[pl.ds(base, CHUNK)], dt_v)

    zeros = jnp.zeros((16,), _f32)

    head = seg_v[pl.ds(0, 16)]
    tail = seg_v[pl.ds(CHUNK - 8, 16)]
    fid = head[8]
    lid = tail[15]

    def zbody(i, _):
        praw_v[pl.ds(fid + i * 16, 16)] = zeros
        return 0
    lax.fori_loop(0, (lid - fid) // 16 + 1, zbody, 0)

    def body(v, carry):
        csum, hsv, slv = carry
        off = v * 16
        s16 = sig_v[pl.ds(off, 16)] * dt_v[pl.ds(off, 16)]
        s_v[pl.ds(off, 16)] = s16
        incl = plsc.cumsum(s16)
        lx = csum + (incl - s16)
        l_v[pl.ds(off, 16)] = lx
        segc = seg_v[pl.ds(off + 8, 16)]
        segp = seg_v[pl.ds(off + 7, 16)]
        rs = segc != segp
        plsc.store_scatter(praw_v, [segc], lx, mask=rs)
        csum = csum + plsc.cummax(lax.rev(incl, (0,)))
        hsv = jnp.maximum(hsv, rs.astype(_f32))
        slv = jnp.maximum(slv, jnp.where(rs, lx, _f32(-1.0)))
        return csum, hsv, slv

    tot_v, hsv, slv = lax.fori_loop(
        0, NVEC, body, (zeros, zeros, zeros - 1.0), unroll=4)
    tot = tot_v[0]
    hs = jnp.max(hsv)
    slast = jnp.max(slv)

    pltpu.sync_copy(s_v, s_hbm.at[pl.ds(base, CHUNK)])
    pltpu.sync_copy(l_v, l_hbm.at[pl.ds(base, CHUNK)])
    pltpu.sync_copy(praw_v.at[pl.ds(0, NR)], praw_hbm.at[wid])

    pos0 = (head[7] != head[8]).astype(_f32)
    stats = (jnp.where(iota == 0, tot, 0.0)
             + jnp.where(iota == 1, hs, 0.0)
             + jnp.where(iota == 2, slast, 0.0)
             + jnp.where(iota == 3, pos0, 0.0)
             + jnp.where(iota == 4, head[8].astype(_f32), 0.0)
             + jnp.where(iota == 5, tail[15].astype(_f32), 0.0)).astype(_f32)
    stage_v[...] = stats
    pltpu.sync_copy(stage_v, stats_hbm.at[pl.ds(wid * 16, 16)])


def _l2_body(s_hbm, l_hbm, seg_hbm, rgb_hbm, z_hbm, praw_hbm, stats_hbm,
             w_hbm, acc_hbm,
             s_v, l_v, z_v, w_v, rgb_v, seg_v, praw_v, stats_v, acc_v, ex_s):
    wid = _wid()
    base = wid * CHUNK
    iota = _iota16()

    seg_v[pl.ds(0, 16)] = jnp.full((16,), -1, _i32)
    @pl.when(wid > 0)
    def _():
        pltpu.sync_copy(seg_hbm.at[pl.ds(base - 8, 8)], seg_v.at[pl.ds(0, 8)])
    pltpu.sync_copy(seg_hbm.at[pl.ds(base, CHUNK)], seg_v.at[pl.ds(8, CHUNK)])
    seg_v[pl.ds(8 + CHUNK, 16)] = jnp.full((16,), -2, _i32)
    @pl.when(wid < NW - 1)
    def _():
        pltpu.sync_copy(seg_hbm.at[pl.ds(base + CHUNK, 8)],
                        seg_v.at[pl.ds(8 + CHUNK, 8)])
    pltpu.sync_copy(s_hbm.at[pl.ds(base, CHUNK)], s_v)
    pltpu.sync_copy(l_hbm.at[pl.ds(base, CHUNK)], l_v)
    pltpu.sync_copy(z_hbm.at[pl.ds(base, CHUNK)], z_v)
    pltpu.sync_copy(rgb_hbm.at[pl.ds(base * 3, CHUNK * 3)], rgb_v)
    pltpu.sync_copy(praw_hbm.at[wid], praw_v)
    pltpu.sync_copy(stats_hbm, stats_v)

    r0 = iota * 16
    r1 = (iota + 16) * 16
    tot0 = plsc.load_gather(stats_v, [r0])
    tot1 = plsc.load_gather(stats_v, [r1])
    hs0 = plsc.load_gather(stats_v, [r0 + 1])
    hs1 = plsc.load_gather(stats_v, [r1 + 1])
    sl0 = plsc.load_gather(stats_v, [r0 + 2])
    sl1 = plsc.load_gather(stats_v, [r1 + 2])

    bw = (jnp.sum(jnp.where(iota < wid, tot0, 0.0))
          + jnp.sum(jnp.where(iota + 16 < wid, tot1, 0.0)))

    head = seg_v[pl.ds(0, 16)]
    pos0_is_start = head[7] != head[8]
    first_id = head[8]

    @pl.when(jnp.logical_not(pos0_is_start))
    def _():
        c0 = jnp.where((hs0 > 0.5) & (iota < wid), iota, -1)
        c1 = jnp.where((hs1 > 0.5) & (iota + 16 < wid), iota + 16, -1)
        wstar = jnp.maximum(jnp.max(c0), jnp.max(c1))
        bws = (jnp.sum(jnp.where(iota < wstar, tot0, 0.0))
               + jnp.sum(jnp.where(iota + 16 < wstar, tot1, 0.0)))
        slws = (jnp.sum(jnp.where(iota == wstar, sl0, 0.0))
                + jnp.sum(jnp.where(iota + 16 == wstar, sl1, 0.0)))
        ov = (bws + slws) - bw
        plsc.store_scatter(praw_v, [jnp.full((16,), first_id, _i32)],
                           jnp.full((16,), ov, _f32), mask=iota == 0)

    zeros = jnp.zeros((16,), _f32)
    tail = seg_v[pl.ds(CHUNK - 8, 16)]
    lid = tail[15]

    def zbody(i, _):
        zb = (first_id + i * 16) * 8
        for j in range(8):
            acc_v[pl.ds(zb + j * 16, 16)] = zeros
        return 0
    lax.fori_loop(0, (lid - first_id) // 16 + 1, zbody, 0)

    def body(v, carries):
        off = v * 16
        s16 = s_v[pl.ds(off, 16)]
        l16 = l_v[pl.ds(off, 16)]
        z16 = z_v[pl.ds(off, 16)]
        segc = seg_v[pl.ds(off + 8, 16)]
        segp = seg_v[pl.ds(off + 7, 16)]
        segn = seg_v[pl.ds(off + 9, 16)]
        idx3 = (off + iota) * 3
        rr = plsc.load_gather(rgb_v, [idx3])
        gg = plsc.load_gather(rgb_v, [idx3 + 1])
        bb = plsc.load_gather(rgb_v, [idx3 + 2])

        pv = plsc.load_gather(praw_v, [segc])
        excl = l16 - pv
        trans = jnp.exp(-excl)
        alpha = 1.0 - jnp.exp(-s16)
        wg = trans * alpha
        w_v[pl.ds(off, 16)] = wg

        rs = segc != segp
        re = segc != segn
        s_lane1 = plsc.cummax(jnp.where(rs, iota + 1, 0))
        fr = s_lane1 == 0
        sidx = jnp.maximum(s_lane1 - 1, 0)
        fr15v = plsc.cummax(lax.rev(s_lane1, (0,))) == 0
        re15v = (plsc.cummax(lax.rev(segc, (0,)))
                 != plsc.cummax(lax.rev(segn, (0,))))

        outs = []
        for c, (q, (cb, cstart)) in enumerate(
                zip((s16, wg * rr, wg * gg, wg * bb, wg * z16), carries)):
            inclq = plsc.cumsum(q)
            exq = inclq - q
            ex_s[pl.ds(c * 16, 16)] = exq
            sxg = plsc.load_gather(ex_s, [sidx + c * 16])
            startex = jnp.where(fr, 0.0, sxg)
            cstart_abs = jnp.where(fr, cstart, cb + sxg)
            runsum = (cb + inclq) - cstart_abs
            plsc.addupdate_scatter(acc_v, [segc * 8 + c], runsum, mask=re)
            sx15 = plsc.cummax(lax.rev(startex, (0,)))
            cb_n = cb + plsc.cummax(lax.rev(inclq, (0,)))
            cstart_n = jnp.where(
                re15v, cb_n, jnp.where(fr15v, cstart, cb + sx15))
            outs.append((cb_n, cstart_n))
        return tuple(outs)

    zv = jnp.zeros((16,), _f32)
    fcarries = lax.fori_loop(
        0, NVEC, body, ((zv, zv),) * 5, unroll=4)

    tail = seg_v[pl.ds(CHUNK - 8, 16)]
    last_id = tail[15]
    lane0 = _iota16() == 0
    for c, (cb, cstart) in enumerate(fcarries):
        fidx = jnp.full((16,), last_id * 8 + c, _i32)
        plsc.addupdate_scatter(acc_v, [fidx], cb - cstart, mask=lane0)

    pltpu.sync_copy(w_v, w_hbm.at[pl.ds(base, CHUNK)])
    pltpu.sync_copy(acc_v.at[pl.ds(0, NR * 8)], acc_hbm.at[wid])


def _l3_body(acc_hbm, tex_hbm, flag_hbm, stats_hbm,
             rgb_hbm, dep_hbm, bg_hbm,
             tmp_v, accsum_v, tex_v, flag_v, rgb_s, dep_s, bg_s, stats_v):
    wid = _wid()
    r0 = wid * RPW
    iota = _iota16()

    for j in range(5 * RPW // 16):
        accsum_v[pl.ds(j * 16, 16)] = jnp.zeros((16,), _f32)

    pltpu.sync_copy(tex_hbm.at[pl.ds(r0, RPW)], tex_v)
    pltpu.sync_copy(flag_hbm, flag_v)
    pltpu.sync_copy(stats_hbm, stats_v)

    mylo = (wid * RPW).astype(_f32)
    myhi = mylo + _f32(RPW - 1)

    rayf = [(r0 + g * 16 + iota).astype(_f32) for g in range(RPW // 16)]

    def body(k, _):
        srow = stats_v[pl.ds(k * 16, 16)]
        fid = srow[4]
        lid = srow[5]

        @pl.when((fid <= myhi) & (lid >= mylo))
        def _():
            pltpu.sync_copy(acc_hbm.at[k, pl.ds(r0 * 8, RPW * 8)], tmp_v)
            for g in range(RPW // 16):
                valid = (rayf[g] >= fid) & (rayf[g] <= lid)
                for c in range(5):
                    idx = (iota + g * 16) * 8 + c
                    val = plsc.load_gather(tmp_v, [idx])
                    sl = pl.ds(c * RPW + g * 16, 16)
                    accsum_v[sl] = accsum_v[sl] + jnp.where(valid, val, 0.0)
        return 0

    lax.fori_loop(0, NW, body, 0)

    flag = flag_v[pl.ds(0, 16)]
    for g in range(RPW // 16):
        sl = pl.ds(g * 16, 16)
        tot = accsum_v[pl.ds(0 * RPW + g * 16, 16)]
        wr = accsum_v[pl.ds(1 * RPW + g * 16, 16)]
        wgr = accsum_v[pl.ds(2 * RPW + g * 16, 16)]
        wb = accsum_v[pl.ds(3 * RPW + g * 16, 16)]
        wz = accsum_v[pl.ds(4 * RPW + g * 16, 16)]
        bg = jnp.exp(-tot)
        dep = wz + flag * bg * tex_v[sl]
        bg_s[sl] = bg
        dep_s[sl] = dep
        ridx = (iota + g * 16) * 3
        plsc.store_scatter(rgb_s, [ridx], wr)
        plsc.store_scatter(rgb_s, [ridx + 1], wgr)
        plsc.store_scatter(rgb_s, [ridx + 2], wb)

    pltpu.sync_copy(bg_s, bg_hbm.at[pl.ds(r0, RPW)])
    pltpu.sync_copy(dep_s, dep_hbm.at[pl.ds(r0, RPW)])
    pltpu.sync_copy(rgb_s, rgb_hbm.at[pl.ds(r0 * 3, RPW * 3)])


def kernel(rgb_samples, radiance_samples, ray_samples_z, ray_samples_dt,
           ray_t_exit, segment_ids, use_ray_t_exit):
    sigma = radiance_samples.reshape(TOTAL)
    rgb_flat = rgb_samples.reshape(TOTAL * 3)
    tex = ray_t_exit.reshape(NR)
    seg = segment_ids.astype(_i32)
    flag16 = jnp.broadcast_to(
        (use_ray_t_exit != 0).astype(_f32), (16,))

    mesh = _mesh()

    l1 = functools.partial(
        pl.kernel,
        out_type=(
            jax.ShapeDtypeStruct((TOTAL,), _f32),
            jax.ShapeDtypeStruct((TOTAL,), _f32),
            jax.ShapeDtypeStruct((NW, NR), _f32),
            jax.ShapeDtypeStruct((NW * 16,), _f32),
        ),
        mesh=mesh,
        compiler_params=_CPARAMS,
        scratch_types=[
            pltpu.VMEM((CHUNK,), _f32),
            pltpu.VMEM((CHUNK,), _f32),
            pltpu.VMEM((CHUNK,), _f32),
            pltpu.VMEM((CHUNK,), _f32),
            pltpu.VMEM((CHUNK + 8,), _i32),
            pltpu.VMEM((NR + 16,), _f32),
            pltpu.VMEM((16,), _f32),
        ],
    )(_l1_body)
    s_arr, l_arr, praw, stats = l1(sigma, ray_samples_dt, seg)

    l2 = functools.partial(
        pl.kernel,
        out_type=(
            jax.ShapeDtypeStruct((TOTAL,), _f32),
            jax.ShapeDtypeStruct((NW, NR * 8), _f32),
        ),
        mesh=mesh,
        compiler_params=_CPARAMS,
        scratch_types=[
            pltpu.VMEM((CHUNK,), _f32),
            pltpu.VMEM((CHUNK,), _f32),
            pltpu.VMEM((CHUNK,), _f32),
            pltpu.VMEM((CHUNK,), _f32),
            pltpu.VMEM((CHUNK * 3,), _f32),
            pltpu.VMEM((CHUNK + 24,), _i32),
            pltpu.VMEM((NR,), _f32),
            pltpu.VMEM((NW * 16,), _f32),
            pltpu.VMEM(((NR + 16) * 8,), _f32),
            pltpu.VMEM((5 * 16,), _f32),
        ],
    )(_l2_body)
    w_arr, acc = l2(s_arr, l_arr, seg, rgb_flat, ray_samples_z, praw, stats)

    l3 = functools.partial(
        pl.kernel,
        out_type=(
            jax.ShapeDtypeStruct((NR * 3,), _f32),
            jax.ShapeDtypeStruct((NR,), _f32),
            jax.ShapeDtypeStruct((NR,), _f32),
        ),
        mesh=mesh,
        compiler_params=_CPARAMS,
        scratch_types=[
            pltpu.VMEM((RPW * 8,), _f32),
            pltpu.VMEM((5 * RPW,), _f32),
            pltpu.VMEM((RPW,), _f32),
            pltpu.VMEM((16,), _f32),
            pltpu.VMEM((RPW * 3,), _f32),
            pltpu.VMEM((RPW,), _f32),
            pltpu.VMEM((RPW,), _f32),
            pltpu.VMEM((NW * 16,), _f32),
        ],
    )(_l3_body)
    rgb_out, dep, bg = l3(acc, tex, flag16, stats)

    return (rgb_out.reshape(NR, 3), dep.reshape(NR, 1), bg.reshape(NR, 1),
            w_arr.reshape(TOTAL, 1))
